# Initial kernel scaffold; baseline (speedup 1.0000x reference)
import jax
import jax.numpy as jnp
from jax.experimental import pallas as pl


def kernel(boxes, scores):
    raise NotImplementedError("write your pallas kernel here")


# speedup vs baseline: 26.6511x; 26.6511x over previous
import jax
import jax.numpy as jnp
from jax.experimental import pallas as pl
from jax.experimental.pallas import tpu as pltpu

_N = 20000
_LANES = 128
_ROWS = 160
_NPAD = _ROWS * _LANES
_MAX_OUT = 300
_NEG = -1e30


def _nms_body(x1_ref, y1_ref, x2_ref, y2_ref, scores_ref,
              ox1_ref, oy1_ref, ox2_ref, oy2_ref, osc_ref,
              live_ref, area_ref):
    x1 = x1_ref[:]
    y1 = y1_ref[:]
    x2 = x2_ref[:]
    y2 = y2_ref[:]
    area_ref[:] = (x2 - x1) * (y2 - y1)
    live_ref[:] = scores_ref[:]

    rowi = jax.lax.broadcasted_iota(jnp.int32, (_ROWS, _LANES), 0)
    coli = jax.lax.broadcasted_iota(jnp.int32, (_ROWS, _LANES), 1)
    fiota = rowi * _LANES + coli
    lane1 = jax.lax.broadcasted_iota(jnp.int32, (1, _LANES), 1)

    def body(t, carry):
        live = live_ref[:]
        m = jnp.max(live)
        idx = jnp.min(jnp.where(live == m, fiota, jnp.int32(_NPAD)))
        r = idx // _LANES
        c = idx % _LANES
        lm = lane1 == c

        def ext(ref):
            row = ref[pl.ds(r, 1), :]
            return jnp.max(jnp.where(lm, row, _NEG))

        bx1 = ext(x1_ref)
        by1 = ext(y1_ref)
        bx2 = ext(x2_ref)
        by2 = ext(y2_ref)
        a1 = (bx2 - bx1) * (by2 - by1)

        ix1 = jnp.maximum(bx1, x1_ref[:])
        iy1 = jnp.maximum(by1, y1_ref[:])
        ix2 = jnp.minimum(bx2, x2_ref[:])
        iy2 = jnp.minimum(by2, y2_ref[:])
        inter = jnp.maximum(ix2 - ix1, 0.0) * jnp.maximum(iy2 - iy1, 0.0)
        denom = a1 + area_ref[:] - inter + jnp.float32(1e-9)
        sup = (inter + inter > denom) | (fiota == idx)
        live_ref[:] = jnp.where(sup, _NEG, live)

        valid = (m > jnp.float32(-0.5e30)).astype(jnp.float32)
        ox1_ref[pl.ds(t, 1), :] = (bx1 * valid).reshape(1, 1)
        oy1_ref[pl.ds(t, 1), :] = (by1 * valid).reshape(1, 1)
        ox2_ref[pl.ds(t, 1), :] = (bx2 * valid).reshape(1, 1)
        oy2_ref[pl.ds(t, 1), :] = (by2 * valid).reshape(1, 1)
        osc_ref[pl.ds(t, 1), :] = (m * valid).reshape(1, 1)
        return carry

    jax.lax.fori_loop(0, _MAX_OUT, body, 0, unroll=False)


def _run_nms(x1, y1, x2, y2, live0, interpret=False):
    out = pl.pallas_call(
        _nms_body,
        out_shape=tuple(
            jax.ShapeDtypeStruct((_MAX_OUT, 1), jnp.float32) for _ in range(5)
        ),
        scratch_shapes=[
            pltpu.VMEM((_ROWS, _LANES), jnp.float32),
            pltpu.VMEM((_ROWS, _LANES), jnp.float32),
        ],
        interpret=interpret,
    )(x1, y1, x2, y2, live0)
    return out


def kernel(boxes, scores):
    pad = _NPAD - _N
    planes = []
    for k in range(4):
        planes.append(jnp.pad(boxes[:, k], (0, pad)).reshape(_ROWS, _LANES))
    live0 = jnp.pad(scores, (0, pad), constant_values=_NEG).reshape(_ROWS, _LANES)
    ox1, oy1, ox2, oy2, osc = _run_nms(*planes, live0)
    return jnp.concatenate([ox1, oy1, ox2, oy2, osc], axis=1)


# speedup vs baseline: 29.0072x; 1.0884x over previous
import jax
import jax.numpy as jnp
from jax.experimental import pallas as pl
from jax.experimental.pallas import tpu as pltpu

_N = 20000
_LANES = 128
_ROWS = 160
_NPAD = _ROWS * _LANES
_MAX_OUT = 300
_NEG = -1e30


def _nms_body(x1_ref, y1_ref, x2_ref, y2_ref, scores_ref,
              ox1_ref, oy1_ref, ox2_ref, oy2_ref, osc_ref,
              live_ref, area_ref, fiota_ref):
    x1 = x1_ref[:]
    y1 = y1_ref[:]
    x2 = x2_ref[:]
    y2 = y2_ref[:]
    area_ref[:] = (x2 - x1) * (y2 - y1)
    live_ref[:] = scores_ref[:]

    rowi = jax.lax.broadcasted_iota(jnp.int32, (_ROWS, _LANES), 0)
    coli = jax.lax.broadcasted_iota(jnp.int32, (_ROWS, _LANES), 1)
    fiota_ref[:] = (rowi * _LANES + coli).astype(jnp.float32)

    m0 = jnp.max(scores_ref[:])

    def body(t, m):
        live = live_ref[:]
        fiota = fiota_ref[:]
        mask2 = live == m
        cnt = jnp.sum(jnp.where(mask2, 1.0, 0.0))
        idxf = jnp.min(jnp.where(mask2, fiota, jnp.float32(1e9)))

        def fast_ext(_):
            bx1 = jnp.max(jnp.where(mask2, x1_ref[:], _NEG))
            by1 = jnp.max(jnp.where(mask2, y1_ref[:], _NEG))
            bx2 = jnp.max(jnp.where(mask2, x2_ref[:], _NEG))
            by2 = jnp.max(jnp.where(mask2, y2_ref[:], _NEG))
            return bx1, by1, bx2, by2

        def tie_ext(_):
            mask3 = mask2 & (fiota == idxf)
            bx1 = jnp.max(jnp.where(mask3, x1_ref[:], _NEG))
            by1 = jnp.max(jnp.where(mask3, y1_ref[:], _NEG))
            bx2 = jnp.max(jnp.where(mask3, x2_ref[:], _NEG))
            by2 = jnp.max(jnp.where(mask3, y2_ref[:], _NEG))
            return bx1, by1, bx2, by2

        bx1, by1, bx2, by2 = jax.lax.cond(cnt > 1.5, tie_ext, fast_ext, 0)

        a1 = (bx2 - bx1) * (by2 - by1)
        valid = (m > jnp.float32(-0.5e30)).astype(jnp.float32)
        ox1_ref[pl.ds(t, 1), :] = (bx1 * valid).reshape(1, 1)
        oy1_ref[pl.ds(t, 1), :] = (by1 * valid).reshape(1, 1)
        ox2_ref[pl.ds(t, 1), :] = (bx2 * valid).reshape(1, 1)
        oy2_ref[pl.ds(t, 1), :] = (by2 * valid).reshape(1, 1)
        osc_ref[pl.ds(t, 1), :] = (m * valid).reshape(1, 1)

        ix1 = jnp.maximum(bx1, x1_ref[:])
        iy1 = jnp.maximum(by1, y1_ref[:])
        ix2 = jnp.minimum(bx2, x2_ref[:])
        iy2 = jnp.minimum(by2, y2_ref[:])
        inter = jnp.maximum(ix2 - ix1, 0.0) * jnp.maximum(iy2 - iy1, 0.0)
        denom = (a1 + area_ref[:]) - inter + jnp.float32(1e-9)
        live_new = jnp.where(inter + inter > denom, _NEG, live)
        live_ref[:] = live_new
        return jnp.max(live_new)

    jax.lax.fori_loop(0, _MAX_OUT, body, m0, unroll=False)


def _run_nms(x1, y1, x2, y2, live0, interpret=False):
    out = pl.pallas_call(
        _nms_body,
        out_shape=tuple(
            jax.ShapeDtypeStruct((_MAX_OUT, 1), jnp.float32) for _ in range(5)
        ),
        scratch_shapes=[
            pltpu.VMEM((_ROWS, _LANES), jnp.float32),
            pltpu.VMEM((_ROWS, _LANES), jnp.float32),
            pltpu.VMEM((_ROWS, _LANES), jnp.float32),
        ],
        interpret=interpret,
    )(x1, y1, x2, y2, live0)
    return out


def kernel(boxes, scores):
    pad = _NPAD - _N
    planes = []
    for k in range(4):
        planes.append(jnp.pad(boxes[:, k], (0, pad)).reshape(_ROWS, _LANES))
    live0 = jnp.pad(scores, (0, pad), constant_values=_NEG).reshape(_ROWS, _LANES)
    ox1, oy1, ox2, oy2, osc = _run_nms(*planes, live0)
    return jnp.concatenate([ox1, oy1, ox2, oy2, osc], axis=1)


# speedup vs baseline: 38.4453x; 1.3254x over previous
import jax
import jax.numpy as jnp
from jax.experimental import pallas as pl
from jax.experimental.pallas import tpu as pltpu

_N = 20000
_LANES = 128
_ROWS = 160
_NPAD = _ROWS * _LANES
_MAX_OUT = 300
_NEG = -1e30


def _nms_body(x1_ref, y1_ref, x2_ref, y2_ref, scores_ref,
              ox1_ref, oy1_ref, ox2_ref, oy2_ref, osc_ref,
              live_ref, area_ref, fiota_ref):
    x1 = x1_ref[:]
    y1 = y1_ref[:]
    x2 = x2_ref[:]
    y2 = y2_ref[:]
    area_ref[:] = (x2 - x1) * (y2 - y1)
    live_ref[:] = scores_ref[:]

    rowi = jax.lax.broadcasted_iota(jnp.int32, (_ROWS, _LANES), 0)
    coli = jax.lax.broadcasted_iota(jnp.int32, (_ROWS, _LANES), 1)
    fiota_ref[:] = (rowi * _LANES + coli).astype(jnp.float32)

    m0 = jnp.max(scores_ref[:])

    def body(t, m):
        live = live_ref[:]
        fiota = fiota_ref[:]
        mask2 = live == m
        cnt = jnp.sum(jnp.where(mask2, 1.0, 0.0))
        idxf = jnp.min(jnp.where(mask2, fiota, jnp.float32(1e9)))

        fx1 = jnp.max(jnp.where(mask2, x1_ref[:], _NEG))
        fy1 = jnp.max(jnp.where(mask2, y1_ref[:], _NEG))
        fx2 = jnp.max(jnp.where(mask2, x2_ref[:], _NEG))
        fy2 = jnp.max(jnp.where(mask2, y2_ref[:], _NEG))

        def tie_ext(_):
            mask3 = mask2 & (fiota == idxf)
            bx1 = jnp.max(jnp.where(mask3, x1_ref[:], _NEG))
            by1 = jnp.max(jnp.where(mask3, y1_ref[:], _NEG))
            bx2 = jnp.max(jnp.where(mask3, x2_ref[:], _NEG))
            by2 = jnp.max(jnp.where(mask3, y2_ref[:], _NEG))
            return bx1, by1, bx2, by2

        def no_tie(_):
            return fx1, fy1, fx2, fy2

        bx1, by1, bx2, by2 = jax.lax.cond(cnt > 1.5, tie_ext, no_tie, 0)

        a1 = (bx2 - bx1) * (by2 - by1)
        valid = (m > jnp.float32(-0.5e30)).astype(jnp.float32)
        ox1_ref[pl.ds(t, 1), :] = (bx1 * valid).reshape(1, 1)
        oy1_ref[pl.ds(t, 1), :] = (by1 * valid).reshape(1, 1)
        ox2_ref[pl.ds(t, 1), :] = (bx2 * valid).reshape(1, 1)
        oy2_ref[pl.ds(t, 1), :] = (by2 * valid).reshape(1, 1)
        osc_ref[pl.ds(t, 1), :] = (m * valid).reshape(1, 1)

        ix1 = jnp.maximum(bx1, x1_ref[:])
        iy1 = jnp.maximum(by1, y1_ref[:])
        ix2 = jnp.minimum(bx2, x2_ref[:])
        iy2 = jnp.minimum(by2, y2_ref[:])
        inter = jnp.maximum(ix2 - ix1, 0.0) * jnp.maximum(iy2 - iy1, 0.0)
        denom = (a1 + area_ref[:]) - inter + jnp.float32(1e-9)
        live_new = jnp.where(inter + inter > denom, _NEG, live)
        live_ref[:] = live_new
        return jnp.max(live_new)

    jax.lax.fori_loop(0, _MAX_OUT, body, m0, unroll=False)


def _run_nms(x1, y1, x2, y2, live0, interpret=False):
    out = pl.pallas_call(
        _nms_body,
        out_shape=tuple(
            jax.ShapeDtypeStruct((_MAX_OUT, 1), jnp.float32) for _ in range(5)
        ),
        scratch_shapes=[
            pltpu.VMEM((_ROWS, _LANES), jnp.float32),
            pltpu.VMEM((_ROWS, _LANES), jnp.float32),
            pltpu.VMEM((_ROWS, _LANES), jnp.float32),
        ],
        interpret=interpret,
    )(x1, y1, x2, y2, live0)
    return out


def kernel(boxes, scores):
    pad = _NPAD - _N
    planes = []
    for k in range(4):
        planes.append(jnp.pad(boxes[:, k], (0, pad)).reshape(_ROWS, _LANES))
    live0 = jnp.pad(scores, (0, pad), constant_values=_NEG).reshape(_ROWS, _LANES)
    ox1, oy1, ox2, oy2, osc = _run_nms(*planes, live0)
    return jnp.concatenate([ox1, oy1, ox2, oy2, osc], axis=1)


# speedup vs baseline: 40.8509x; 1.0626x over previous
import jax
import jax.numpy as jnp
from jax.experimental import pallas as pl
from jax.experimental.pallas import tpu as pltpu

_N = 20000
_LANES = 128
_ROWS = 160
_NPAD = _ROWS * _LANES
_MAX_OUT = 300
_NEG = -1e30
_BIGF = 1e9


def _exact_pick(live, fiota, x1_ref, y1_ref, x2_ref, y2_ref):
    m = jnp.max(live)
    eq = live == m
    idxf = jnp.min(jnp.where(eq, fiota, _BIGF))
    mk = eq & (fiota == idxf)
    bx1 = jnp.max(jnp.where(mk, x1_ref[:], _NEG))
    by1 = jnp.max(jnp.where(mk, y1_ref[:], _NEG))
    bx2 = jnp.max(jnp.where(mk, x2_ref[:], _NEG))
    by2 = jnp.max(jnp.where(mk, y2_ref[:], _NEG))
    return m, bx1, by1, bx2, by2


def _nms_body(x1_ref, y1_ref, x2_ref, y2_ref, scores_ref,
              ox1_ref, oy1_ref, ox2_ref, oy2_ref, osc_ref,
              live_ref, area_ref, fiota_ref):
    x1 = x1_ref[:]
    y1 = y1_ref[:]
    x2 = x2_ref[:]
    y2 = y2_ref[:]
    area_ref[:] = (x2 - x1) * (y2 - y1)
    live_ref[:] = scores_ref[:]

    rowi = jax.lax.broadcasted_iota(jnp.int32, (_ROWS, _LANES), 0)
    coli = jax.lax.broadcasted_iota(jnp.int32, (_ROWS, _LANES), 1)
    fiota_ref[:] = (rowi * _LANES + coli).astype(jnp.float32)

    carry0 = _exact_pick(scores_ref[:], fiota_ref[:],
                         x1_ref, y1_ref, x2_ref, y2_ref)

    def body(t, carry):
        m, bx1, by1, bx2, by2 = carry
        live = live_ref[:]

        valid = (m > jnp.float32(-0.5e30)).astype(jnp.float32)
        ox1_ref[pl.ds(t, 1), :] = (bx1 * valid).reshape(1, 1)
        oy1_ref[pl.ds(t, 1), :] = (by1 * valid).reshape(1, 1)
        ox2_ref[pl.ds(t, 1), :] = (bx2 * valid).reshape(1, 1)
        oy2_ref[pl.ds(t, 1), :] = (by2 * valid).reshape(1, 1)
        osc_ref[pl.ds(t, 1), :] = (m * valid).reshape(1, 1)

        iseq = live == m
        cnt = jnp.sum(jnp.where(iseq, 1.0, 0.0))
        excl = jnp.where(iseq, _NEG, live)
        m2 = jnp.max(excl)
        mru = excl == m2
        cnt2 = jnp.sum(jnp.where(mru, 1.0, 0.0))
        cx1 = jnp.max(jnp.where(mru, x1_ref[:], _NEG))
        cy1 = jnp.max(jnp.where(mru, y1_ref[:], _NEG))
        cx2 = jnp.max(jnp.where(mru, x2_ref[:], _NEG))
        cy2 = jnp.max(jnp.where(mru, y2_ref[:], _NEG))

        a1 = (bx2 - bx1) * (by2 - by1)
        ix1 = jnp.maximum(bx1, x1_ref[:])
        iy1 = jnp.maximum(by1, y1_ref[:])
        ix2 = jnp.minimum(bx2, x2_ref[:])
        iy2 = jnp.minimum(by2, y2_ref[:])
        inter = jnp.maximum(ix2 - ix1, 0.0) * jnp.maximum(iy2 - iy1, 0.0)
        denom = (a1 + area_ref[:]) - inter + jnp.float32(1e-9)
        live_ref[:] = jnp.where(inter + inter > denom, _NEG, live)

        rx1 = jnp.maximum(bx1, cx1)
        ry1 = jnp.maximum(by1, cy1)
        rx2 = jnp.minimum(bx2, cx2)
        ry2 = jnp.minimum(by2, cy2)
        rint = jnp.maximum(rx2 - rx1, 0.0) * jnp.maximum(ry2 - ry1, 0.0)
        ca = (cx2 - cx1) * (cy2 - cy1)
        rden = (a1 + ca) - rint + jnp.float32(1e-9)
        ru_sup = rint + rint > rden

        rare = ((cnt > 1.5) | (cnt2 > 1.5)
                | (m2 <= jnp.float32(-0.5e30)) | ru_sup)

        def rare_fn(_):
            return _exact_pick(live_ref[:], fiota_ref[:],
                               x1_ref, y1_ref, x2_ref, y2_ref)

        def common_fn(_):
            return m2, cx1, cy1, cx2, cy2

        return jax.lax.cond(rare, rare_fn, common_fn, 0)

    jax.lax.fori_loop(0, _MAX_OUT, body, carry0, unroll=False)


def _run_nms(x1, y1, x2, y2, live0, interpret=False):
    out = pl.pallas_call(
        _nms_body,
        out_shape=tuple(
            jax.ShapeDtypeStruct((_MAX_OUT, 1), jnp.float32) for _ in range(5)
        ),
        scratch_shapes=[
            pltpu.VMEM((_ROWS, _LANES), jnp.float32),
            pltpu.VMEM((_ROWS, _LANES), jnp.float32),
            pltpu.VMEM((_ROWS, _LANES), jnp.float32),
        ],
        interpret=interpret,
    )(x1, y1, x2, y2, live0)
    return out


def kernel(boxes, scores):
    pad = _NPAD - _N
    planes = []
    for k in range(4):
        planes.append(jnp.pad(boxes[:, k], (0, pad)).reshape(_ROWS, _LANES))
    live0 = jnp.pad(scores, (0, pad), constant_values=_NEG).reshape(_ROWS, _LANES)
    ox1, oy1, ox2, oy2, osc = _run_nms(*planes, live0)
    return jnp.concatenate([ox1, oy1, ox2, oy2, osc], axis=1)


# speedup vs baseline: 65.4292x; 1.6017x over previous
import jax
import jax.numpy as jnp
from jax.experimental import pallas as pl
from jax.experimental.pallas import tpu as pltpu

_N = 20000
_LANES = 128
_ROWS = 160
_NPAD = _ROWS * _LANES
_MAX_OUT = 300
_NEG = -1e30
_BIGF = 1e9


def _exact_pick(live, fiota, x1_ref, y1_ref, x2_ref, y2_ref):
    m = jnp.max(live)
    eq = live == m
    cnt = jnp.sum(jnp.where(eq, 1.0, 0.0))
    idxf = jnp.min(jnp.where(eq, fiota, _BIGF))
    mk = eq & (fiota == idxf)
    bx1 = jnp.max(jnp.where(mk, x1_ref[:], _NEG))
    by1 = jnp.max(jnp.where(mk, y1_ref[:], _NEG))
    bx2 = jnp.max(jnp.where(mk, x2_ref[:], _NEG))
    by2 = jnp.max(jnp.where(mk, y2_ref[:], _NEG))
    return m, bx1, by1, bx2, by2, cnt


def _nms_body(x1_ref, y1_ref, x2_ref, y2_ref, scores_ref,
              ox1_ref, oy1_ref, ox2_ref, oy2_ref, osc_ref,
              live_ref, area_ref, fiota_ref):
    x1 = x1_ref[:]
    y1 = y1_ref[:]
    x2 = x2_ref[:]
    y2 = y2_ref[:]
    area_ref[:] = (x2 - x1) * (y2 - y1)
    live_ref[:] = scores_ref[:]

    rowi = jax.lax.broadcasted_iota(jnp.int32, (_ROWS, _LANES), 0)
    coli = jax.lax.broadcasted_iota(jnp.int32, (_ROWS, _LANES), 1)
    fiota_ref[:] = (rowi * _LANES + coli).astype(jnp.float32)
    fiota = fiota_ref[:]

    live0 = scores_ref[:]
    m0, b0x1, b0y1, b0x2, b0y2, cnt0 = _exact_pick(
        live0, fiota, x1_ref, y1_ref, x2_ref, y2_ref)
    mn0 = jnp.max(jnp.where(live0 == m0, _NEG, live0))
    carry0 = (m0, b0x1, b0y1, b0x2, b0y2,
              mn0, cnt0 > 1.5)

    def body(t, carry):
        m, bx1, by1, bx2, by2, mn, force = carry
        live = live_ref[:]

        valid = (m > jnp.float32(-0.5e30)).astype(jnp.float32)
        ox1_ref[pl.ds(t, 1), :] = (bx1 * valid).reshape(1, 1)
        oy1_ref[pl.ds(t, 1), :] = (by1 * valid).reshape(1, 1)
        ox2_ref[pl.ds(t, 1), :] = (bx2 * valid).reshape(1, 1)
        oy2_ref[pl.ds(t, 1), :] = (by2 * valid).reshape(1, 1)
        osc_ref[pl.ds(t, 1), :] = (m * valid).reshape(1, 1)

        mn_eq = live == mn
        cntn = jnp.sum(jnp.where(mn_eq, 1.0, 0.0))
        nx1 = jnp.max(jnp.where(mn_eq, x1_ref[:], _NEG))
        ny1 = jnp.max(jnp.where(mn_eq, y1_ref[:], _NEG))
        nx2 = jnp.max(jnp.where(mn_eq, x2_ref[:], _NEG))
        ny2 = jnp.max(jnp.where(mn_eq, y2_ref[:], _NEG))
        excl = jnp.where(live == m, _NEG, jnp.where(mn_eq, _NEG, live))
        m3 = jnp.max(excl)

        a1 = (bx2 - bx1) * (by2 - by1)
        ix1 = jnp.maximum(bx1, x1_ref[:])
        iy1 = jnp.maximum(by1, y1_ref[:])
        ix2 = jnp.minimum(bx2, x2_ref[:])
        iy2 = jnp.minimum(by2, y2_ref[:])
        inter = jnp.maximum(ix2 - ix1, 0.0) * jnp.maximum(iy2 - iy1, 0.0)
        denom = (a1 + area_ref[:]) - inter + jnp.float32(1e-9)
        live_ref[:] = jnp.where(inter + inter > denom, _NEG, live)

        jx1 = jnp.maximum(bx1, nx1)
        jy1 = jnp.maximum(by1, ny1)
        jx2 = jnp.minimum(bx2, nx2)
        jy2 = jnp.minimum(by2, ny2)
        jint = jnp.maximum(jx2 - jx1, 0.0) * jnp.maximum(jy2 - jy1, 0.0)
        na = (nx2 - nx1) * (ny2 - ny1)
        jden = (a1 + na) - jint + jnp.float32(1e-9)
        sup_cur = jint + jint > jden

        rare = (force | (cntn < 0.5) | (cntn > 1.5)
                | (mn <= jnp.float32(-0.5e30)) | sup_cur)

        def rare_fn(_):
            ln = live_ref[:]
            em, ex1, ey1, ex2, ey2, ecnt = _exact_pick(
                ln, fiota, x1_ref, y1_ref, x2_ref, y2_ref)
            emn = jnp.max(jnp.where(ln == em, _NEG, ln))
            return (em, ex1, ey1, ex2, ey2, emn, ecnt > 1.5)

        def common_fn(_):
            return (mn, nx1, ny1, nx2, ny2, m3, jnp.bool_(False))

        return jax.lax.cond(rare, rare_fn, common_fn, 0)

    jax.lax.fori_loop(0, _MAX_OUT, body, carry0, unroll=False)


def _run_nms(x1, y1, x2, y2, live0, interpret=False):
    out = pl.pallas_call(
        _nms_body,
        out_shape=tuple(
            jax.ShapeDtypeStruct((_MAX_OUT, 1), jnp.float32) for _ in range(5)
        ),
        scratch_shapes=[
            pltpu.VMEM((_ROWS, _LANES), jnp.float32),
            pltpu.VMEM((_ROWS, _LANES), jnp.float32),
            pltpu.VMEM((_ROWS, _LANES), jnp.float32),
        ],
        interpret=interpret,
    )(x1, y1, x2, y2, live0)
    return out


def kernel(boxes, scores):
    pad = _NPAD - _N
    planes = []
    for k in range(4):
        planes.append(jnp.pad(boxes[:, k], (0, pad)).reshape(_ROWS, _LANES))
    live0 = jnp.pad(scores, (0, pad), constant_values=_NEG).reshape(_ROWS, _LANES)
    ox1, oy1, ox2, oy2, osc = _run_nms(*planes, live0)
    return jnp.concatenate([ox1, oy1, ox2, oy2, osc], axis=1)


# speedup vs baseline: 65.7899x; 1.0055x over previous
import jax
import jax.numpy as jnp
from jax.experimental import pallas as pl
from jax.experimental.pallas import tpu as pltpu

_N = 20000
_LANES = 128
_ROWS = 160
_NPAD = _ROWS * _LANES
_MAX_OUT = 300
_NEG = -1e30
_BIGF = 1e9


def _exact_pick(live, fiota, x1_ref, y1_ref, x2_ref, y2_ref):
    m = jnp.max(live)
    eq = live == m
    cnt = jnp.sum(jnp.where(eq, 1.0, 0.0))
    idxf = jnp.min(jnp.where(eq, fiota, _BIGF))
    mk = eq & (fiota == idxf)
    bx1 = jnp.max(jnp.where(mk, x1_ref[:], _NEG))
    by1 = jnp.max(jnp.where(mk, y1_ref[:], _NEG))
    bx2 = jnp.max(jnp.where(mk, x2_ref[:], _NEG))
    by2 = jnp.max(jnp.where(mk, y2_ref[:], _NEG))
    return m, bx1, by1, bx2, by2, cnt


def _iou_gt_half(a, b):
    ax1, ay1, ax2, ay2 = a
    bx1, by1, bx2, by2 = b
    ix1 = jnp.maximum(ax1, bx1)
    iy1 = jnp.maximum(ay1, by1)
    ix2 = jnp.minimum(ax2, bx2)
    iy2 = jnp.minimum(ay2, by2)
    inter = jnp.maximum(ix2 - ix1, 0.0) * jnp.maximum(iy2 - iy1, 0.0)
    aa = (ax2 - ax1) * (ay2 - ay1)
    ab = (bx2 - bx1) * (by2 - by1)
    denom = (aa + ab) - inter + jnp.float32(1e-9)
    return inter + inter > denom


def _nms_body(x1_ref, y1_ref, x2_ref, y2_ref, scores_ref,
              ox1_ref, oy1_ref, ox2_ref, oy2_ref, osc_ref,
              live_ref, area_ref, fiota_ref):
    x1 = x1_ref[:]
    y1 = y1_ref[:]
    x2 = x2_ref[:]
    y2 = y2_ref[:]
    area_ref[:] = (x2 - x1) * (y2 - y1)
    live_ref[:] = scores_ref[:]

    rowi = jax.lax.broadcasted_iota(jnp.int32, (_ROWS, _LANES), 0)
    coli = jax.lax.broadcasted_iota(jnp.int32, (_ROWS, _LANES), 1)
    fiota_ref[:] = (rowi * _LANES + coli).astype(jnp.float32)
    fiota = fiota_ref[:]

    def suppress(live, box, a1):
        bx1, by1, bx2, by2 = box
        ix1 = jnp.maximum(bx1, x1_ref[:])
        iy1 = jnp.maximum(by1, y1_ref[:])
        ix2 = jnp.minimum(bx2, x2_ref[:])
        iy2 = jnp.minimum(by2, y2_ref[:])
        inter = jnp.maximum(ix2 - ix1, 0.0) * jnp.maximum(iy2 - iy1, 0.0)
        denom = (a1 + area_ref[:]) - inter + jnp.float32(1e-9)
        return jnp.where(inter + inter > denom, _NEG, live)

    def emit(t, m, box, valid):
        bx1, by1, bx2, by2 = box
        ox1_ref[pl.ds(t, 1), :] = (bx1 * valid).reshape(1, 1)
        oy1_ref[pl.ds(t, 1), :] = (by1 * valid).reshape(1, 1)
        ox2_ref[pl.ds(t, 1), :] = (bx2 * valid).reshape(1, 1)
        oy2_ref[pl.ds(t, 1), :] = (by2 * valid).reshape(1, 1)
        osc_ref[pl.ds(t, 1), :] = (m * valid).reshape(1, 1)

    live0 = scores_ref[:]
    m0, p0x1, p0y1, p0x2, p0y2, cnt0 = _exact_pick(
        live0, fiota, x1_ref, y1_ref, x2_ref, y2_ref)
    e0 = jnp.where(live0 == m0, _NEG, live0)
    v10 = jnp.max(e0)
    v20 = jnp.max(jnp.where(e0 == v10, _NEG, e0))
    carry0 = (m0, p0x1, p0y1, p0x2, p0y2,
              v10, v20, cnt0 > 1.5)

    def body(i, carry):
        m, px1, py1, px2, py2, v1, v2, force = carry
        t = i * 2
        live = live_ref[:]
        pbox = (px1, py1, px2, py2)

        validp = (m > jnp.float32(-0.5e30)).astype(jnp.float32)
        emit(t, m, pbox, validp)

        mA = live == v1
        cntA = jnp.sum(jnp.where(mA, 1.0, 0.0))
        ax1 = jnp.max(jnp.where(mA, x1_ref[:], _NEG))
        ay1 = jnp.max(jnp.where(mA, y1_ref[:], _NEG))
        ax2 = jnp.max(jnp.where(mA, x2_ref[:], _NEG))
        ay2 = jnp.max(jnp.where(mA, y2_ref[:], _NEG))
        mB = live == v2
        cntB = jnp.sum(jnp.where(mB, 1.0, 0.0))
        bx1 = jnp.max(jnp.where(mB, x1_ref[:], _NEG))
        by1 = jnp.max(jnp.where(mB, y1_ref[:], _NEG))
        bx2 = jnp.max(jnp.where(mB, x2_ref[:], _NEG))
        by2 = jnp.max(jnp.where(mB, y2_ref[:], _NEG))
        excl = jnp.where(live == m, _NEG,
                         jnp.where(mA, _NEG, jnp.where(mB, _NEG, live)))
        v3 = jnp.max(excl)

        a1 = (px2 - px1) * (py2 - py1)
        live1 = suppress(live, pbox, a1)
        live_ref[:] = live1

        abox = (ax1, ay1, ax2, ay2)
        bbox = (bx1, by1, bx2, by2)
        bad_a = ((cntA < 0.5) | (cntA > 1.5)
                 | (v1 <= jnp.float32(-0.5e30)) | _iou_gt_half(pbox, abox))
        bad_b = ((cntB < 0.5) | (cntB > 1.5)
                 | (v2 <= jnp.float32(-0.5e30)) | _iou_gt_half(pbox, bbox)
                 | _iou_gt_half(abox, bbox))
        rare = force | bad_a | bad_b

        def rare_fn(_):
            ln1 = live_ref[:]
            em, ex1, ey1, ex2, ey2, ecnt = _exact_pick(
                ln1, fiota, x1_ref, y1_ref, x2_ref, y2_ref)
            ebox = (ex1, ey1, ex2, ey2)
            evalid = (em > jnp.float32(-0.5e30)).astype(jnp.float32)
            emit(t + 1, em, ebox, evalid)
            ea = (ex2 - ex1) * (ey2 - ey1)
            ln2 = suppress(ln1, ebox, ea)
            live_ref[:] = ln2
            nm, nx1, ny1, nx2, ny2, ncnt = _exact_pick(
                ln2, fiota, x1_ref, y1_ref, x2_ref, y2_ref)
            ne = jnp.where(ln2 == nm, _NEG, ln2)
            nv1 = jnp.max(ne)
            nv2 = jnp.max(jnp.where(ne == nv1, _NEG, ne))
            return (nm, nx1, ny1, nx2, ny2, nv1, nv2, ncnt > 1.5)

        def common_fn(_):
            v4 = jnp.max(jnp.where(excl == v3, _NEG, excl))
            valida = (v1 > jnp.float32(-0.5e30)).astype(jnp.float32)
            emit(t + 1, v1, abox, valida)
            aa = (ax2 - ax1) * (ay2 - ay1)
            live_ref[:] = suppress(live1, abox, aa)
            return (v2, bx1, by1, bx2, by2, v3, v4, jnp.bool_(False))

        return jax.lax.cond(rare, rare_fn, common_fn, 0)

    jax.lax.fori_loop(0, _MAX_OUT // 2, body, carry0, unroll=False)


def _run_nms(x1, y1, x2, y2, live0, interpret=False):
    out = pl.pallas_call(
        _nms_body,
        out_shape=tuple(
            jax.ShapeDtypeStruct((_MAX_OUT, 1), jnp.float32) for _ in range(5)
        ),
        scratch_shapes=[
            pltpu.VMEM((_ROWS, _LANES), jnp.float32),
            pltpu.VMEM((_ROWS, _LANES), jnp.float32),
            pltpu.VMEM((_ROWS, _LANES), jnp.float32),
        ],
        interpret=interpret,
    )(x1, y1, x2, y2, live0)
    return out


def kernel(boxes, scores):
    pad = _NPAD - _N
    planes = []
    for k in range(4):
        planes.append(jnp.pad(boxes[:, k], (0, pad)).reshape(_ROWS, _LANES))
    live0 = jnp.pad(scores, (0, pad), constant_values=_NEG).reshape(_ROWS, _LANES)
    ox1, oy1, ox2, oy2, osc = _run_nms(*planes, live0)
    return jnp.concatenate([ox1, oy1, ox2, oy2, osc], axis=1)
